# baseline (speedup 1.0000x reference)
import functools

import jax
import jax.numpy as jnp
from jax import lax
from jax.experimental import pallas as pl
from jax.experimental.pallas import tpu as pltpu
from jax.experimental.pallas import tpu_sc as plsc

NC = 2
NS = 16
LANES = 16
NW = NC * NS


def _vector_mesh():
    return plsc.VectorSubcoreMesh(core_axis_name="c", subcore_axis_name="s")


_SC_PARAMS = pltpu.CompilerParams(use_tc_tiling_on_sc=False)
_SC_PARAMS_NOLAYOUT = pltpu.CompilerParams(use_tc_tiling_on_sc=False,
                                           needs_layout_passes=False)


def _padded_rows(n):
    return ((n + NS * 8 - 1) // (NS * 8)) * (NS * 8)


def _sc_counts(eidx, n):
    _, _, C, K = eidx.shape
    np_ = _padded_rows(n)
    rpt = np_ // NS

    @functools.partial(
        pl.kernel,
        out_type=jax.ShapeDtypeStruct((NC, np_, LANES), jnp.float32),
        mesh=_vector_mesh(),
        compiler_params=_SC_PARAMS,
        scratch_types=[
            pltpu.VMEM((C, K), jnp.int32),
            pltpu.VMEM((K, LANES), jnp.float32),
            pltpu.VMEM((rpt, LANES), jnp.float32),
            pltpu.VMEM_SHARED((np_, LANES), jnp.float32),
        ],
    )
    def k(eidx_hbm, out_hbm, idx_v, ones_v, zero_v, acc_s):
        cid = lax.axis_index("c")
        sid = lax.axis_index("s")
        wid = sid * NC + cid

        pltpu.sync_copy(eidx_hbm.at[1, wid], idx_v)

        @pl.loop(0, K)
        def _(i):
            ones_v[i] = jnp.ones((LANES,), jnp.float32)

        @pl.loop(0, rpt)
        def _(i):
            zero_v[i] = jnp.zeros((LANES,), jnp.float32)

        base = sid * rpt
        pltpu.sync_copy(zero_v, acc_s.at[pl.ds(base, rpt)])
        plsc.subcore_barrier()

        @pl.loop(0, C)
        def _(j):
            pltpu.sync_copy(ones_v, acc_s.at[idx_v.at[j]], add=True)

        plsc.subcore_barrier()
        pltpu.sync_copy(acc_s.at[pl.ds(base, rpt)],
                        out_hbm.at[cid, pl.ds(base, rpt)])

    return k(eidx)


def _sc_agg(table, eidx):
    n, h = table.shape
    _, _, C, K = eidx.shape
    np_ = _padded_rows(n)
    rpt = np_ // NS

    nbuf = 4
    assert (C - 1) % nbuf == 0

    @functools.partial(
        pl.kernel,
        out_type=jax.ShapeDtypeStruct((NC, np_, h), jnp.float32),
        mesh=_vector_mesh(),
        compiler_params=_SC_PARAMS,
        scratch_types=[
            pltpu.VMEM((2, C, K), jnp.int32),
            pltpu.VMEM((nbuf, K, h), jnp.float32),
            pltpu.VMEM((rpt, h), jnp.float32),
            pltpu.VMEM_SHARED((np_, h), jnp.float32),
            [pltpu.SemaphoreType.DMA] * nbuf,
            [pltpu.SemaphoreType.DMA] * nbuf,
            pltpu.SemaphoreType.DMA,
        ],
    )
    def k(table_hbm, eidx_hbm, out_hbm, idx_v, bufs, zero_v,
          acc_s, gsems, ssems, isem):
        cid = lax.axis_index("c")
        sid = lax.axis_index("s")
        wid = sid * NC + cid

        cp_s = pltpu.async_copy(eidx_hbm.at[0, wid], idx_v.at[0], isem)
        cp_d = pltpu.async_copy(eidx_hbm.at[1, wid], idx_v.at[1], isem)

        @pl.loop(0, rpt)
        def _(i):
            @pl.loop(0, h, step=LANES)
            def _(c):
                zero_v[i, pl.ds(c, LANES)] = jnp.zeros((LANES,), jnp.float32)

        base = sid * rpt
        cp_s.wait()
        cp_d.wait()
        pltpu.sync_copy(zero_v, acc_s.at[pl.ds(base, rpt)])
        plsc.subcore_barrier()

        def gather(j, b):
            pltpu.async_copy(table_hbm.at[idx_v.at[0, j]], bufs.at[b],
                             gsems[b])

        def wait_gather(b):
            pltpu.make_async_copy(table_hbm.at[idx_v.at[0, 0]], bufs.at[b],
                                  gsems[b]).wait()

        def scatter(j, b):
            pltpu.async_copy(bufs.at[b], acc_s.at[idx_v.at[1, j]], ssems[b],
                             add=True)

        def wait_scatter(b):
            pltpu.make_async_copy(bufs.at[b], acc_s.at[idx_v.at[1, 0]],
                                  ssems[b]).wait()

        for b in range(nbuf):
            gather(b, b)

        @pl.loop(0, (C - 1) // nbuf)
        def _(m):
            j0 = nbuf * m
            for b in range(nbuf):
                j = j0 + b
                wait_gather(b)
                scatter(j, b)

                @pl.when(j + nbuf < C)
                def _():
                    wait_scatter(b)
                    gather(j + nbuf, b)

        wait_gather(0)
        scatter(C - 1, 0)
        for b in range(nbuf):
            wait_scatter(b)

        plsc.subcore_barrier()
        pltpu.sync_copy(acc_s.at[pl.ds(base, rpt)],
                        out_hbm.at[cid, pl.ds(base, rpt)])

    return k(table, eidx)


def _sc_decode(a_tab, b_tab, wvec, biasvec, eidx):
    n, h = a_tab.shape
    _, _, C, K = eidx.shape
    ew = C * K
    e = NW * ew
    nslice = h // LANES
    ngrp = K // LANES
    nbuf = 4
    assert (C - 1) % nbuf == 0

    @functools.partial(
        pl.kernel,
        out_type=jax.ShapeDtypeStruct((e,), jnp.float32),
        mesh=_vector_mesh(),
        compiler_params=_SC_PARAMS_NOLAYOUT,
        scratch_types=[
            pltpu.VMEM((2, C, K), jnp.int32),
            pltpu.VMEM((nbuf, K, h), jnp.float32),
            pltpu.VMEM((nbuf, K, h), jnp.float32),
            pltpu.VMEM((nbuf, K), jnp.float32),
            pltpu.VMEM((h,), jnp.float32),
            pltpu.VMEM((LANES,), jnp.float32),
            [pltpu.SemaphoreType.DMA] * (2 * nbuf),
        ],
    )
    def k(a_hbm, b_hbm, w_hbm, bias_hbm, eidx_hbm, out_hbm, idx_v,
          bufs_a, bufs_b, out_v, w_v, bias_v, gsems):
        cid = lax.axis_index("c")
        sid = lax.axis_index("s")
        wid = sid * NC + cid

        pltpu.sync_copy(eidx_hbm.at[0, wid], idx_v.at[0])
        pltpu.sync_copy(eidx_hbm.at[1, wid], idx_v.at[1])
        pltpu.sync_copy(w_hbm.at[0], w_v)
        pltpu.sync_copy(bias_hbm, bias_v)

        ws = [w_v[pl.ds(c * LANES, LANES)] for c in range(nslice)]
        bias = bias_v[...]
        lane = lax.iota(jnp.int32, LANES)
        masks = [lane == li for li in range(LANES)]

        def gather(j, slot):
            pltpu.async_copy(a_hbm.at[idx_v.at[0, j]], bufs_a.at[slot],
                             gsems[slot])
            pltpu.async_copy(b_hbm.at[idx_v.at[1, j]], bufs_b.at[slot],
                             gsems[nbuf + slot])

        def wait_gather(slot):
            pltpu.make_async_copy(a_hbm.at[idx_v.at[0, 0]], bufs_a.at[slot],
                                  gsems[slot]).wait()
            pltpu.make_async_copy(b_hbm.at[idx_v.at[1, 0]], bufs_b.at[slot],
                                  gsems[nbuf + slot]).wait()

        def compute_store(j, slot):
            a_v = bufs_a.at[slot]
            b_v = bufs_b.at[slot]
            o_v = out_v.at[slot]

            @pl.loop(0, ngrp)
            def _(g):
                logits = bias
                for li in range(LANES):
                    ei = g * LANES + li
                    acc = bias
                    for c in range(nslice):
                        s = (a_v[ei, pl.ds(c * LANES, LANES)]
                             + b_v[ei, pl.ds(c * LANES, LANES)])
                        acc = acc + ws[c] * jnp.maximum(s, 0.0)
                    tot = jnp.broadcast_to(jnp.sum(acc), (LANES,))
                    logits = jnp.where(masks[li], tot, logits)
                o_v[pl.ds(g * LANES, LANES)] = 1.0 / (1.0 + jnp.exp(-logits))

            pltpu.sync_copy(o_v, out_hbm.at[pl.ds(wid * ew + j * K, K)])

        for b in range(nbuf):
            gather(b, b)

        @pl.loop(0, (C - 1) // nbuf)
        def _(m):
            j0 = nbuf * m
            for b in range(nbuf):
                j = j0 + b
                wait_gather(b)
                compute_store(j, b)

                @pl.when(j + nbuf < C)
                def _():
                    gather(j + nbuf, b)

        wait_gather(0)
        compute_store(C - 1, 0)

    return k(a_tab, b_tab, wvec, biasvec, eidx)


def _tc_mm(xx, ww, bn=1000):
    n, d = xx.shape
    h = ww.shape[1]

    def body(x_ref, w_ref, o_ref):
        o_ref[...] = jnp.dot(x_ref[...], w_ref[...],
                             preferred_element_type=jnp.float32)

    return pl.pallas_call(
        body,
        grid=(n // bn,),
        in_specs=[pl.BlockSpec((bn, d), lambda i: (i, 0)),
                  pl.BlockSpec((d, h), lambda i: (0, 0))],
        out_specs=pl.BlockSpec((bn, h), lambda i: (i, 0)),
        out_shape=jax.ShapeDtypeStruct((n, h), jnp.float32),
    )(xx, ww)


def _tc_scale(h1, cnt, bn=1000):
    n, h = h1.shape

    def body(h_ref, c_ref, o_ref, dinv_ref):
        deg = 1.0 + c_ref[0, :, 0:1] + c_ref[1, :, 0:1]
        dinv = lax.rsqrt(deg)
        o_ref[...] = h_ref[...] * dinv
        dinv_ref[...] = dinv

    return pl.pallas_call(
        body,
        grid=(n // bn,),
        in_specs=[pl.BlockSpec((bn, h), lambda i: (i, 0)),
                  pl.BlockSpec((NC, bn, LANES), lambda i: (0, i, 0))],
        out_specs=[pl.BlockSpec((bn, h), lambda i: (i, 0)),
                   pl.BlockSpec((bn, 1), lambda i: (i, 0))],
        out_shape=[jax.ShapeDtypeStruct((n, h), jnp.float32),
                   jax.ShapeDtypeStruct((n, 1), jnp.float32)],
    )(h1, cnt)


def _tc_layer(agg, hs, dinv, bias, ww, relu, bn=1000):
    n, h = hs.shape

    def body(a_ref, hs_ref, dinv_ref, b_ref, w_ref, o_ref):
        z = dinv_ref[...] * (a_ref[0] + a_ref[1] + hs_ref[...]) + b_ref[...]
        if relu:
            z = jnp.maximum(z, 0.0)
        o_ref[...] = jnp.dot(z, w_ref[...],
                             preferred_element_type=jnp.float32) * dinv_ref[...]

    return pl.pallas_call(
        body,
        grid=(n // bn,),
        in_specs=[pl.BlockSpec((NC, bn, h), lambda i: (0, i, 0)),
                  pl.BlockSpec((bn, h), lambda i: (i, 0)),
                  pl.BlockSpec((bn, 1), lambda i: (i, 0)),
                  pl.BlockSpec((1, h), lambda i: (0, 0)),
                  pl.BlockSpec((h, h), lambda i: (0, 0))],
        out_specs=pl.BlockSpec((bn, h), lambda i: (i, 0)),
        out_shape=jax.ShapeDtypeStruct((n, h), jnp.float32),
    )(agg, hs, dinv, bias, ww)


def _tc_decode_tables(agg, hs, dinv, bias, wa, wb, l1b, bn=1000):
    n, h = hs.shape

    def body(a_ref, hs_ref, dinv_ref, b_ref, wa_ref, wb_ref, l1b_ref,
             oa_ref, ob_ref):
        z = dinv_ref[...] * (a_ref[0] + a_ref[1] + hs_ref[...]) + b_ref[...]
        oa_ref[...] = jnp.dot(z, wa_ref[...],
                              preferred_element_type=jnp.float32) + l1b_ref[...]
        ob_ref[...] = jnp.dot(z, wb_ref[...],
                              preferred_element_type=jnp.float32)

    return pl.pallas_call(
        body,
        grid=(n // bn,),
        in_specs=[pl.BlockSpec((NC, bn, h), lambda i: (0, i, 0)),
                  pl.BlockSpec((bn, h), lambda i: (i, 0)),
                  pl.BlockSpec((bn, 1), lambda i: (i, 0)),
                  pl.BlockSpec((1, h), lambda i: (0, 0)),
                  pl.BlockSpec((h, h), lambda i: (0, 0)),
                  pl.BlockSpec((h, h), lambda i: (0, 0)),
                  pl.BlockSpec((1, h), lambda i: (0, 0))],
        out_specs=[pl.BlockSpec((bn, h), lambda i: (i, 0)),
                   pl.BlockSpec((bn, h), lambda i: (i, 0))],
        out_shape=[jax.ShapeDtypeStruct((n, h), jnp.float32),
                   jax.ShapeDtypeStruct((n, h), jnp.float32)],
    )(agg, hs, dinv, bias, wa, wb, l1b)


def _tc_decode(asrc, bdst, w2, b2, bn=4000):
    e, h = asrc.shape

    def body(a_ref, b_ref, w_ref, bb_ref, o_ref):
        z = jnp.maximum(a_ref[...] + b_ref[...], 0.0)
        o_ref[...] = jax.nn.sigmoid(
            jnp.dot(z, w_ref[...], preferred_element_type=jnp.float32)
            + bb_ref[...])

    return pl.pallas_call(
        body,
        grid=(e // bn,),
        in_specs=[pl.BlockSpec((bn, h), lambda i: (i, 0)),
                  pl.BlockSpec((bn, h), lambda i: (i, 0)),
                  pl.BlockSpec((h, 1), lambda i: (0, 0)),
                  pl.BlockSpec((1, 1), lambda i: (0, 0))],
        out_specs=pl.BlockSpec((bn, 1), lambda i: (i, 0)),
        out_shape=jax.ShapeDtypeStruct((e, 1), jnp.float32),
    )(asrc, bdst, w2, b2)


def kernel(x, edge_index, W1, b1, W2, b2, lin1_W, lin1_b, lin2_W, lin2_b):
    n, _ = x.shape
    h = W1.shape[1]
    e = edge_index.shape[1]

    ew = e // NW
    K = 80
    C = ew // K
    assert ew == C * K and C % 2 == 1

    eidx = edge_index.reshape(2, NW, C, K)

    cnt = _sc_counts(eidx, n)
    h1 = _tc_mm(x, W1)
    h1s, dinv = _tc_scale(h1, cnt)

    agg1 = _sc_agg(h1s, eidx)
    h2s = _tc_layer(agg1, h1s, dinv, b1.reshape(1, h), W2, relu=True)

    agg2 = _sc_agg(h2s, eidx)
    a_tab, b_tab = _tc_decode_tables(agg2, h2s, dinv, b2.reshape(1, h),
                                     lin1_W[:h], lin1_W[h:],
                                     lin1_b.reshape(1, h))

    wvec = lin2_W.reshape(1, h)
    biasvec = jnp.full((LANES,), lin2_b[0] / LANES, dtype=jnp.float32)
    out = _sc_decode(a_tab, b_tab, wvec, biasvec, eidx)
    return out.reshape(e, 1)


# speedup vs baseline: 1.1788x; 1.1788x over previous
import functools

import jax
import jax.numpy as jnp
from jax import lax
from jax.experimental import pallas as pl
from jax.experimental.pallas import tpu as pltpu
from jax.experimental.pallas import tpu_sc as plsc

NC = 2
NS = 16
LANES = 16
NW = NC * NS


def _vector_mesh():
    return plsc.VectorSubcoreMesh(core_axis_name="c", subcore_axis_name="s")


_SC_PARAMS = pltpu.CompilerParams(use_tc_tiling_on_sc=False)
_SC_PARAMS_NOLAYOUT = pltpu.CompilerParams(use_tc_tiling_on_sc=False,
                                           needs_layout_passes=False)


def _padded_rows(n):
    return ((n + NS * 8 - 1) // (NS * 8)) * (NS * 8)


def _sc_counts(eidx, n):
    _, _, C, K = eidx.shape
    np_ = _padded_rows(n)
    rpt = np_ // NS

    @functools.partial(
        pl.kernel,
        out_type=jax.ShapeDtypeStruct((NC, np_, LANES), jnp.float32),
        mesh=_vector_mesh(),
        compiler_params=_SC_PARAMS,
        scratch_types=[
            pltpu.VMEM((C, K), jnp.int32),
            pltpu.VMEM((K, LANES), jnp.float32),
            pltpu.VMEM((rpt, LANES), jnp.float32),
            pltpu.VMEM_SHARED((np_, LANES), jnp.float32),
        ],
    )
    def k(eidx_hbm, out_hbm, idx_v, ones_v, zero_v, acc_s):
        cid = lax.axis_index("c")
        sid = lax.axis_index("s")
        wid = sid * NC + cid

        pltpu.sync_copy(eidx_hbm.at[1, wid], idx_v)

        @pl.loop(0, K)
        def _(i):
            ones_v[i] = jnp.ones((LANES,), jnp.float32)

        @pl.loop(0, rpt)
        def _(i):
            zero_v[i] = jnp.zeros((LANES,), jnp.float32)

        base = sid * rpt
        pltpu.sync_copy(zero_v, acc_s.at[pl.ds(base, rpt)])
        plsc.subcore_barrier()

        @pl.loop(0, C)
        def _(j):
            pltpu.sync_copy(ones_v, acc_s.at[idx_v.at[j]], add=True)

        plsc.subcore_barrier()
        pltpu.sync_copy(acc_s.at[pl.ds(base, rpt)],
                        out_hbm.at[cid, pl.ds(base, rpt)])

    return k(eidx)


def _sc_agg(table, eidx):
    n, h = table.shape
    _, _, C, K = eidx.shape
    np_ = _padded_rows(n)
    rpt = np_ // NS

    nbuf = 4
    assert (C - 1) % nbuf == 0

    @functools.partial(
        pl.kernel,
        out_type=jax.ShapeDtypeStruct((NC, np_, h), jnp.float32),
        mesh=_vector_mesh(),
        compiler_params=_SC_PARAMS,
        scratch_types=[
            pltpu.VMEM((2, C, K), jnp.int32),
            pltpu.VMEM((nbuf, K, h), jnp.float32),
            pltpu.VMEM((rpt, h), jnp.float32),
            pltpu.VMEM_SHARED((np_, h), jnp.float32),
            [pltpu.SemaphoreType.DMA] * nbuf,
            [pltpu.SemaphoreType.DMA] * nbuf,
            pltpu.SemaphoreType.DMA,
        ],
    )
    def k(table_hbm, eidx_hbm, out_hbm, idx_v, bufs, zero_v,
          acc_s, gsems, ssems, isem):
        cid = lax.axis_index("c")
        sid = lax.axis_index("s")
        wid = sid * NC + cid

        cp_s = pltpu.async_copy(eidx_hbm.at[0, wid], idx_v.at[0], isem)
        cp_d = pltpu.async_copy(eidx_hbm.at[1, wid], idx_v.at[1], isem)

        @pl.loop(0, rpt)
        def _(i):
            @pl.loop(0, h, step=LANES)
            def _(c):
                zero_v[i, pl.ds(c, LANES)] = jnp.zeros((LANES,), jnp.float32)

        base = sid * rpt
        cp_s.wait()
        cp_d.wait()
        pltpu.sync_copy(zero_v, acc_s.at[pl.ds(base, rpt)])
        plsc.subcore_barrier()

        def gather(j, b):
            pltpu.async_copy(table_hbm.at[idx_v.at[0, j]], bufs.at[b],
                             gsems[b])

        def wait_gather(b):
            pltpu.make_async_copy(table_hbm.at[idx_v.at[0, 0]], bufs.at[b],
                                  gsems[b]).wait()

        def scatter(j, b):
            pltpu.async_copy(bufs.at[b], acc_s.at[idx_v.at[1, j]], ssems[b],
                             add=True)

        def wait_scatter(b):
            pltpu.make_async_copy(bufs.at[b], acc_s.at[idx_v.at[1, 0]],
                                  ssems[b]).wait()

        for b in range(nbuf):
            gather(b, b)

        @pl.loop(0, (C - 1) // nbuf)
        def _(m):
            j0 = nbuf * m
            for b in range(nbuf):
                j = j0 + b
                wait_gather(b)
                scatter(j, b)

                @pl.when(j + nbuf < C)
                def _():
                    wait_scatter(b)
                    gather(j + nbuf, b)

        wait_gather(0)
        scatter(C - 1, 0)
        for b in range(nbuf):
            wait_scatter(b)

        plsc.subcore_barrier()
        pltpu.sync_copy(acc_s.at[pl.ds(base, rpt)],
                        out_hbm.at[cid, pl.ds(base, rpt)])

    return k(table, eidx)


def _sc_decode(a_tab, b_tab, wvec, biasvec, eidx):
    n, h = a_tab.shape
    _, _, C, K = eidx.shape
    ew = C * K
    e = NW * ew
    nslice = h // LANES
    ngrp = K // LANES
    nbuf = 2
    assert (C - 1) % nbuf == 0

    @functools.partial(
        pl.kernel,
        out_type=jax.ShapeDtypeStruct((e,), jnp.float32),
        mesh=_vector_mesh(),
        compiler_params=_SC_PARAMS_NOLAYOUT,
        scratch_types=[
            pltpu.VMEM((2, C, K), jnp.int32),
            pltpu.VMEM((nbuf, K, h), jnp.float32),
            pltpu.VMEM((nbuf, K, h), jnp.float32),
            pltpu.VMEM((nbuf, K), jnp.float32),
            pltpu.VMEM((h,), jnp.float32),
            pltpu.VMEM((LANES,), jnp.float32),
            [pltpu.SemaphoreType.DMA] * (2 * nbuf),
        ],
    )
    def k(a_hbm, b_hbm, w_hbm, bias_hbm, eidx_hbm, out_hbm, idx_v,
          bufs_a, bufs_b, out_v, w_v, bias_v, gsems):
        cid = lax.axis_index("c")
        sid = lax.axis_index("s")
        wid = sid * NC + cid

        pltpu.sync_copy(eidx_hbm.at[0, wid], idx_v.at[0])
        pltpu.sync_copy(eidx_hbm.at[1, wid], idx_v.at[1])
        pltpu.sync_copy(w_hbm.at[0], w_v)
        pltpu.sync_copy(bias_hbm, bias_v)

        ws = [w_v[pl.ds(c * LANES, LANES)] for c in range(nslice)]
        bias = bias_v[...]
        lane = lax.iota(jnp.int32, LANES)
        masks = [lane == li for li in range(LANES)]

        def gather(j, slot):
            pltpu.async_copy(a_hbm.at[idx_v.at[0, j]], bufs_a.at[slot],
                             gsems[slot])
            pltpu.async_copy(b_hbm.at[idx_v.at[1, j]], bufs_b.at[slot],
                             gsems[nbuf + slot])

        def wait_gather(slot):
            pltpu.make_async_copy(a_hbm.at[idx_v.at[0, 0]], bufs_a.at[slot],
                                  gsems[slot]).wait()
            pltpu.make_async_copy(b_hbm.at[idx_v.at[1, 0]], bufs_b.at[slot],
                                  gsems[nbuf + slot]).wait()

        def compute_store(j, slot):
            a_v = bufs_a.at[slot]
            b_v = bufs_b.at[slot]
            o_v = out_v.at[slot]

            @pl.loop(0, ngrp)
            def _(g):
                logits = bias
                for li in range(LANES):
                    ei = g * LANES + li
                    acc = bias
                    for c in range(nslice):
                        s = (a_v[ei, pl.ds(c * LANES, LANES)]
                             + b_v[ei, pl.ds(c * LANES, LANES)])
                        acc = acc + ws[c] * jnp.maximum(s, 0.0)
                    tot = jnp.broadcast_to(jnp.sum(acc), (LANES,))
                    logits = jnp.where(masks[li], tot, logits)
                o_v[pl.ds(g * LANES, LANES)] = 1.0 / (1.0 + jnp.exp(-logits))

            pltpu.sync_copy(o_v, out_hbm.at[pl.ds(wid * ew + j * K, K)])

        for b in range(nbuf):
            gather(b, b)

        @pl.loop(0, (C - 1) // nbuf)
        def _(m):
            j0 = nbuf * m
            for b in range(nbuf):
                j = j0 + b
                wait_gather(b)
                compute_store(j, b)

                @pl.when(j + nbuf < C)
                def _():
                    gather(j + nbuf, b)

        wait_gather(0)
        compute_store(C - 1, 0)

    return k(a_tab, b_tab, wvec, biasvec, eidx)


def _tc_mm(xx, ww, bn=1000):
    n, d = xx.shape
    h = ww.shape[1]

    def body(x_ref, w_ref, o_ref):
        o_ref[...] = jnp.dot(x_ref[...], w_ref[...],
                             preferred_element_type=jnp.float32)

    return pl.pallas_call(
        body,
        grid=(n // bn,),
        in_specs=[pl.BlockSpec((bn, d), lambda i: (i, 0)),
                  pl.BlockSpec((d, h), lambda i: (0, 0))],
        out_specs=pl.BlockSpec((bn, h), lambda i: (i, 0)),
        out_shape=jax.ShapeDtypeStruct((n, h), jnp.float32),
    )(xx, ww)


def _tc_scale(h1, cnt, bn=1000):
    n, h = h1.shape

    def body(h_ref, c_ref, o_ref, dinv_ref):
        deg = 1.0 + c_ref[0, :, 0:1] + c_ref[1, :, 0:1]
        dinv = lax.rsqrt(deg)
        o_ref[...] = h_ref[...] * dinv
        dinv_ref[...] = dinv

    return pl.pallas_call(
        body,
        grid=(n // bn,),
        in_specs=[pl.BlockSpec((bn, h), lambda i: (i, 0)),
                  pl.BlockSpec((NC, bn, LANES), lambda i: (0, i, 0))],
        out_specs=[pl.BlockSpec((bn, h), lambda i: (i, 0)),
                   pl.BlockSpec((bn, 1), lambda i: (i, 0))],
        out_shape=[jax.ShapeDtypeStruct((n, h), jnp.float32),
                   jax.ShapeDtypeStruct((n, 1), jnp.float32)],
    )(h1, cnt)


def _tc_layer(agg, hs, dinv, bias, ww, relu, bn=1000):
    n, h = hs.shape

    def body(a_ref, hs_ref, dinv_ref, b_ref, w_ref, o_ref):
        z = dinv_ref[...] * (a_ref[0] + a_ref[1] + hs_ref[...]) + b_ref[...]
        if relu:
            z = jnp.maximum(z, 0.0)
        o_ref[...] = jnp.dot(z, w_ref[...],
                             preferred_element_type=jnp.float32) * dinv_ref[...]

    return pl.pallas_call(
        body,
        grid=(n // bn,),
        in_specs=[pl.BlockSpec((NC, bn, h), lambda i: (0, i, 0)),
                  pl.BlockSpec((bn, h), lambda i: (i, 0)),
                  pl.BlockSpec((bn, 1), lambda i: (i, 0)),
                  pl.BlockSpec((1, h), lambda i: (0, 0)),
                  pl.BlockSpec((h, h), lambda i: (0, 0))],
        out_specs=pl.BlockSpec((bn, h), lambda i: (i, 0)),
        out_shape=jax.ShapeDtypeStruct((n, h), jnp.float32),
    )(agg, hs, dinv, bias, ww)


def _tc_decode_tables(agg, hs, dinv, bias, wa, wb, l1b, bn=1000):
    n, h = hs.shape

    def body(a_ref, hs_ref, dinv_ref, b_ref, wa_ref, wb_ref, l1b_ref,
             oa_ref, ob_ref):
        z = dinv_ref[...] * (a_ref[0] + a_ref[1] + hs_ref[...]) + b_ref[...]
        oa_ref[...] = jnp.dot(z, wa_ref[...],
                              preferred_element_type=jnp.float32) + l1b_ref[...]
        ob_ref[...] = jnp.dot(z, wb_ref[...],
                              preferred_element_type=jnp.float32)

    return pl.pallas_call(
        body,
        grid=(n // bn,),
        in_specs=[pl.BlockSpec((NC, bn, h), lambda i: (0, i, 0)),
                  pl.BlockSpec((bn, h), lambda i: (i, 0)),
                  pl.BlockSpec((bn, 1), lambda i: (i, 0)),
                  pl.BlockSpec((1, h), lambda i: (0, 0)),
                  pl.BlockSpec((h, h), lambda i: (0, 0)),
                  pl.BlockSpec((h, h), lambda i: (0, 0)),
                  pl.BlockSpec((1, h), lambda i: (0, 0))],
        out_specs=[pl.BlockSpec((bn, h), lambda i: (i, 0)),
                   pl.BlockSpec((bn, h), lambda i: (i, 0))],
        out_shape=[jax.ShapeDtypeStruct((n, h), jnp.float32),
                   jax.ShapeDtypeStruct((n, h), jnp.float32)],
    )(agg, hs, dinv, bias, wa, wb, l1b)


def _tc_decode(asrc, bdst, w2, b2, bn=4000):
    e, h = asrc.shape

    def body(a_ref, b_ref, w_ref, bb_ref, o_ref):
        z = jnp.maximum(a_ref[...] + b_ref[...], 0.0)
        o_ref[...] = jax.nn.sigmoid(
            jnp.dot(z, w_ref[...], preferred_element_type=jnp.float32)
            + bb_ref[...])

    return pl.pallas_call(
        body,
        grid=(e // bn,),
        in_specs=[pl.BlockSpec((bn, h), lambda i: (i, 0)),
                  pl.BlockSpec((bn, h), lambda i: (i, 0)),
                  pl.BlockSpec((h, 1), lambda i: (0, 0)),
                  pl.BlockSpec((1, 1), lambda i: (0, 0))],
        out_specs=pl.BlockSpec((bn, 1), lambda i: (i, 0)),
        out_shape=jax.ShapeDtypeStruct((e, 1), jnp.float32),
    )(asrc, bdst, w2, b2)


def kernel(x, edge_index, W1, b1, W2, b2, lin1_W, lin1_b, lin2_W, lin2_b):
    n, _ = x.shape
    h = W1.shape[1]
    e = edge_index.shape[1]

    ew = e // NW
    K = 80
    C = ew // K
    assert ew == C * K and C % 2 == 1

    eidx = edge_index.reshape(2, NW, C, K)

    cnt = _sc_counts(eidx, n)
    h1 = _tc_mm(x, W1)
    h1s, dinv = _tc_scale(h1, cnt)

    agg1 = _sc_agg(h1s, eidx)
    h2s = _tc_layer(agg1, h1s, dinv, b1.reshape(1, h), W2, relu=True)

    agg2 = _sc_agg(h2s, eidx)
    a_tab, b_tab = _tc_decode_tables(agg2, h2s, dinv, b2.reshape(1, h),
                                     lin1_W[:h], lin1_W[h:],
                                     lin1_b.reshape(1, h))

    wvec = lin2_W.reshape(1, h)
    biasvec = jnp.full((LANES,), lin2_b[0] / LANES, dtype=jnp.float32)
    out = _sc_decode(a_tab, b_tab, wvec, biasvec, eidx)
    return out.reshape(e, 1)


# speedup vs baseline: 1.3459x; 1.1417x over previous
import functools

import jax
import jax.numpy as jnp
from jax import lax
from jax.experimental import pallas as pl
from jax.experimental.pallas import tpu as pltpu
from jax.experimental.pallas import tpu_sc as plsc

NC = 2
NS = 16
LANES = 16
NW = NC * NS


def _vector_mesh():
    return plsc.VectorSubcoreMesh(core_axis_name="c", subcore_axis_name="s")


_SC_PARAMS = pltpu.CompilerParams(use_tc_tiling_on_sc=False)
_SC_PARAMS_NOLAYOUT = pltpu.CompilerParams(use_tc_tiling_on_sc=False,
                                           needs_layout_passes=False)


def _padded_rows(n):
    return ((n + NS * 8 - 1) // (NS * 8)) * (NS * 8)


def _sc_counts(eidx, n, h):
    _, _, C, K = eidx.shape
    np_ = _padded_rows(n)
    rpt = np_ // NS

    @functools.partial(
        pl.kernel,
        out_type=jax.ShapeDtypeStruct((NC, np_, h), jnp.float32),
        mesh=_vector_mesh(),
        compiler_params=_SC_PARAMS,
        scratch_types=[
            pltpu.VMEM((C, K), jnp.int32),
            pltpu.VMEM((K, h), jnp.float32),
            pltpu.VMEM((rpt, h), jnp.float32),
            pltpu.VMEM_SHARED((np_, h), jnp.float32),
            pltpu.SemaphoreType.DMA,
            pltpu.SemaphoreType.DMA,
        ],
    )
    def k(eidx_hbm, out_hbm, idx_v, ones_v, zero_v, acc_s, sem0, sem1):
        cid = lax.axis_index("c")
        sid = lax.axis_index("s")
        wid = sid * NC + cid

        pltpu.sync_copy(eidx_hbm.at[1, wid], idx_v)

        @pl.loop(0, K)
        def _(i):
            @pl.loop(0, h, step=LANES)
            def _(c):
                ones_v[i, pl.ds(c, LANES)] = jnp.ones((LANES,), jnp.float32)

        @pl.loop(0, rpt)
        def _(i):
            @pl.loop(0, h, step=LANES)
            def _(c):
                zero_v[i, pl.ds(c, LANES)] = jnp.zeros((LANES,), jnp.float32)

        base = sid * rpt
        pltpu.sync_copy(zero_v, acc_s.at[pl.ds(base, rpt)])
        plsc.subcore_barrier()

        sems = [sem0, sem1]

        def scatter(j, b):
            pltpu.async_copy(ones_v, acc_s.at[idx_v.at[j]], sems[b],
                             add=True)

        def wait_scatter(b):
            pltpu.make_async_copy(ones_v, acc_s.at[idx_v.at[0]],
                                  sems[b]).wait()

        scatter(0, 0)
        scatter(1, 1)

        @pl.loop(0, (C - 1) // 2)
        def _(m):
            j = 2 * m
            for b in range(2):
                wait_scatter(b)

                @pl.when(j + b + 2 < C)
                def _():
                    scatter(j + b + 2, b)

        wait_scatter(0)

        plsc.subcore_barrier()
        pltpu.sync_copy(acc_s.at[pl.ds(base, rpt)],
                        out_hbm.at[cid, pl.ds(base, rpt)])

    return k(eidx)


def _sc_agg(table, eidx):
    n, h = table.shape
    _, _, C, K = eidx.shape
    np_ = _padded_rows(n)
    rpt = np_ // NS

    nbuf = 4
    assert (C - 1) % nbuf == 0

    @functools.partial(
        pl.kernel,
        out_type=jax.ShapeDtypeStruct((NC, np_, h), jnp.float32),
        mesh=_vector_mesh(),
        compiler_params=_SC_PARAMS,
        scratch_types=[
            pltpu.VMEM((2, C, K), jnp.int32),
            pltpu.VMEM((nbuf, K, h), jnp.float32),
            pltpu.VMEM((rpt, h), jnp.float32),
            pltpu.VMEM_SHARED((np_, h), jnp.float32),
            [pltpu.SemaphoreType.DMA] * nbuf,
            [pltpu.SemaphoreType.DMA] * nbuf,
            pltpu.SemaphoreType.DMA,
        ],
    )
    def k(table_hbm, eidx_hbm, out_hbm, idx_v, bufs, zero_v,
          acc_s, gsems, ssems, isem):
        cid = lax.axis_index("c")
        sid = lax.axis_index("s")
        wid = sid * NC + cid

        cp_s = pltpu.async_copy(eidx_hbm.at[0, wid], idx_v.at[0], isem)
        cp_d = pltpu.async_copy(eidx_hbm.at[1, wid], idx_v.at[1], isem)

        @pl.loop(0, rpt)
        def _(i):
            @pl.loop(0, h, step=LANES)
            def _(c):
                zero_v[i, pl.ds(c, LANES)] = jnp.zeros((LANES,), jnp.float32)

        base = sid * rpt
        cp_s.wait()
        cp_d.wait()
        pltpu.sync_copy(zero_v, acc_s.at[pl.ds(base, rpt)])
        plsc.subcore_barrier()

        def gather(j, b):
            pltpu.async_copy(table_hbm.at[idx_v.at[0, j]], bufs.at[b],
                             gsems[b])

        def wait_gather(b):
            pltpu.make_async_copy(table_hbm.at[idx_v.at[0, 0]], bufs.at[b],
                                  gsems[b]).wait()

        def scatter(j, b):
            pltpu.async_copy(bufs.at[b], acc_s.at[idx_v.at[1, j]], ssems[b],
                             add=True)

        def wait_scatter(b):
            pltpu.make_async_copy(bufs.at[b], acc_s.at[idx_v.at[1, 0]],
                                  ssems[b]).wait()

        for b in range(nbuf):
            gather(b, b)

        @pl.loop(0, (C - 1) // nbuf)
        def _(m):
            j0 = nbuf * m
            for b in range(nbuf):
                j = j0 + b
                wait_gather(b)
                scatter(j, b)

                @pl.when(j + nbuf < C)
                def _():
                    wait_scatter(b)
                    gather(j + nbuf, b)

        wait_gather(0)
        scatter(C - 1, 0)
        for b in range(nbuf):
            wait_scatter(b)

        plsc.subcore_barrier()
        pltpu.sync_copy(acc_s.at[pl.ds(base, rpt)],
                        out_hbm.at[cid, pl.ds(base, rpt)])

    return k(table, eidx)


def _sc_decode(a_tab, b_tab, wvec, biasvec, eidx):
    n, h = a_tab.shape
    _, _, C, K = eidx.shape
    ew = C * K
    e = NW * ew
    nslice = h // LANES
    ngrp = K // LANES
    nbuf = 2
    assert (C - 1) % nbuf == 0

    @functools.partial(
        pl.kernel,
        out_type=jax.ShapeDtypeStruct((e,), jnp.float32),
        mesh=_vector_mesh(),
        compiler_params=_SC_PARAMS_NOLAYOUT,
        scratch_types=[
            pltpu.VMEM((2, C, K), jnp.int32),
            pltpu.VMEM((nbuf, K, h), jnp.float32),
            pltpu.VMEM((nbuf, K, h), jnp.float32),
            pltpu.VMEM((nbuf, K), jnp.float32),
            pltpu.VMEM((h,), jnp.float32),
            pltpu.VMEM((LANES,), jnp.float32),
            [pltpu.SemaphoreType.DMA] * (2 * nbuf),
        ],
    )
    def k(a_hbm, b_hbm, w_hbm, bias_hbm, eidx_hbm, out_hbm, idx_v,
          bufs_a, bufs_b, out_v, w_v, bias_v, gsems):
        cid = lax.axis_index("c")
        sid = lax.axis_index("s")
        wid = sid * NC + cid

        pltpu.sync_copy(eidx_hbm.at[0, wid], idx_v.at[0])
        pltpu.sync_copy(eidx_hbm.at[1, wid], idx_v.at[1])
        pltpu.sync_copy(w_hbm.at[0], w_v)
        pltpu.sync_copy(bias_hbm, bias_v)

        ws = [w_v[pl.ds(c * LANES, LANES)] for c in range(nslice)]
        bias = bias_v[...]
        lane = lax.iota(jnp.int32, LANES)
        masks = [lane == li for li in range(LANES)]

        def gather(j, slot):
            pltpu.async_copy(a_hbm.at[idx_v.at[0, j]], bufs_a.at[slot],
                             gsems[slot])
            pltpu.async_copy(b_hbm.at[idx_v.at[1, j]], bufs_b.at[slot],
                             gsems[nbuf + slot])

        def wait_gather(slot):
            pltpu.make_async_copy(a_hbm.at[idx_v.at[0, 0]], bufs_a.at[slot],
                                  gsems[slot]).wait()
            pltpu.make_async_copy(b_hbm.at[idx_v.at[1, 0]], bufs_b.at[slot],
                                  gsems[nbuf + slot]).wait()

        def compute_store(j, slot):
            a_v = bufs_a.at[slot]
            b_v = bufs_b.at[slot]
            o_v = out_v.at[slot]

            @pl.loop(0, ngrp)
            def _(g):
                logits = bias
                for li in range(LANES):
                    ei = g * LANES + li
                    acc = bias
                    for c in range(nslice):
                        s = (a_v[ei, pl.ds(c * LANES, LANES)]
                             + b_v[ei, pl.ds(c * LANES, LANES)])
                        acc = acc + ws[c] * jnp.maximum(s, 0.0)
                    tot = jnp.broadcast_to(jnp.sum(acc), (LANES,))
                    logits = jnp.where(masks[li], tot, logits)
                o_v[pl.ds(g * LANES, LANES)] = 1.0 / (1.0 + jnp.exp(-logits))

            pltpu.sync_copy(o_v, out_hbm.at[pl.ds(wid * ew + j * K, K)])

        for b in range(nbuf):
            gather(b, b)

        @pl.loop(0, (C - 1) // nbuf)
        def _(m):
            j0 = nbuf * m
            for b in range(nbuf):
                j = j0 + b
                wait_gather(b)
                compute_store(j, b)

                @pl.when(j + nbuf < C)
                def _():
                    gather(j + nbuf, b)

        wait_gather(0)
        compute_store(C - 1, 0)

    return k(a_tab, b_tab, wvec, biasvec, eidx)


def _tc_mm(xx, ww, bn=1000):
    n, d = xx.shape
    h = ww.shape[1]

    def body(x_ref, w_ref, o_ref):
        o_ref[...] = jnp.dot(x_ref[...], w_ref[...],
                             preferred_element_type=jnp.float32)

    return pl.pallas_call(
        body,
        grid=(n // bn,),
        in_specs=[pl.BlockSpec((bn, d), lambda i: (i, 0)),
                  pl.BlockSpec((d, h), lambda i: (0, 0))],
        out_specs=pl.BlockSpec((bn, h), lambda i: (i, 0)),
        out_shape=jax.ShapeDtypeStruct((n, h), jnp.float32),
    )(xx, ww)


def _tc_scale(h1p, cnt2, bn=1000):
    n2, w = h1p.shape

    def body(h_ref, c_ref, o_ref, dinv_ref):
        dinv = lax.rsqrt(1.0 + c_ref[0] + c_ref[1])
        o_ref[...] = h_ref[...] * dinv
        dinv_ref[...] = dinv

    return pl.pallas_call(
        body,
        grid=(n2 // bn,),
        in_specs=[pl.BlockSpec((bn, w), lambda i: (i, 0)),
                  pl.BlockSpec((NC, bn, w), lambda i: (0, i, 0))],
        out_specs=[pl.BlockSpec((bn, w), lambda i: (i, 0)),
                   pl.BlockSpec((bn, w), lambda i: (i, 0))],
        out_shape=[jax.ShapeDtypeStruct((n2, w), jnp.float32),
                   jax.ShapeDtypeStruct((n2, w), jnp.float32)],
    )(h1p, cnt2)


def _tc_layer(aggp, hsp, dinvp, b128, wbd, relu, bn=1000):
    n2, w = hsp.shape

    def body(a_ref, hs_ref, dinv_ref, b_ref, w_ref, o_ref):
        dinv = dinv_ref[...]
        z = dinv * (a_ref[0] + a_ref[1] + hs_ref[...]) + b_ref[...]
        if relu:
            z = jnp.maximum(z, 0.0)
        o_ref[...] = jnp.dot(z, w_ref[...],
                             preferred_element_type=jnp.float32) * dinv

    return pl.pallas_call(
        body,
        grid=(n2 // bn,),
        in_specs=[pl.BlockSpec((NC, bn, w), lambda i: (0, i, 0)),
                  pl.BlockSpec((bn, w), lambda i: (i, 0)),
                  pl.BlockSpec((bn, w), lambda i: (i, 0)),
                  pl.BlockSpec((1, w), lambda i: (0, 0)),
                  pl.BlockSpec((w, w), lambda i: (0, 0))],
        out_specs=pl.BlockSpec((bn, w), lambda i: (i, 0)),
        out_shape=jax.ShapeDtypeStruct((n2, w), jnp.float32),
    )(aggp, hsp, dinvp, b128, wbd)


def _tc_decode_tables(aggp, hsp, dinvp, b128, wabd, wbbd, l1b128, bn=1000):
    n2, w = hsp.shape

    def body(a_ref, hs_ref, dinv_ref, b_ref, wa_ref, wb_ref, l1b_ref,
             oa_ref, ob_ref):
        z = dinv_ref[...] * (a_ref[0] + a_ref[1] + hs_ref[...]) + b_ref[...]
        oa_ref[...] = jnp.dot(z, wa_ref[...],
                              preferred_element_type=jnp.float32) + l1b_ref[...]
        ob_ref[...] = jnp.dot(z, wb_ref[...],
                              preferred_element_type=jnp.float32)

    return pl.pallas_call(
        body,
        grid=(n2 // bn,),
        in_specs=[pl.BlockSpec((NC, bn, w), lambda i: (0, i, 0)),
                  pl.BlockSpec((bn, w), lambda i: (i, 0)),
                  pl.BlockSpec((bn, w), lambda i: (i, 0)),
                  pl.BlockSpec((1, w), lambda i: (0, 0)),
                  pl.BlockSpec((w, w), lambda i: (0, 0)),
                  pl.BlockSpec((w, w), lambda i: (0, 0)),
                  pl.BlockSpec((1, w), lambda i: (0, 0))],
        out_specs=[pl.BlockSpec((bn, w), lambda i: (i, 0)),
                   pl.BlockSpec((bn, w), lambda i: (i, 0))],
        out_shape=[jax.ShapeDtypeStruct((n2, w), jnp.float32),
                   jax.ShapeDtypeStruct((n2, w), jnp.float32)],
    )(aggp, hsp, dinvp, b128, wabd, wbbd, l1b128)


def kernel(x, edge_index, W1, b1, W2, b2, lin1_W, lin1_b, lin2_W, lin2_b):
    n, d = x.shape
    h = W1.shape[1]
    e = edge_index.shape[1]

    ew = e // NW
    K = 80
    C = ew // K
    assert ew == C * K and C % 2 == 1

    n2 = n // 2
    np2 = _padded_rows(n) * h // 128

    eidx = edge_index.reshape(2, NW, C, K)

    zdh = jnp.zeros((d, h), jnp.float32)
    zhh = jnp.zeros((h, h), jnp.float32)
    w1a, w1b = lin1_W[:h], lin1_W[h:]
    w1bd = jnp.block([[W1, zdh], [zdh, W1]])
    w2bd = jnp.block([[W2, zhh], [zhh, W2]])
    wabd = jnp.block([[w1a, zhh], [zhh, w1a]])
    wbbd = jnp.block([[w1b, zhh], [zhh, w1b]])
    b1p = jnp.concatenate([b1, b1]).reshape(1, 2 * h)
    b2p = jnp.concatenate([b2, b2]).reshape(1, 2 * h)
    l1bp = jnp.concatenate([lin1_b, lin1_b]).reshape(1, 2 * h)

    cnt2 = _sc_counts(eidx, n, h).reshape(NC, np2, 128)
    h1p = _tc_mm(x.reshape(n2, 2 * d), w1bd, bn=1000)
    h1sp, dinvp = _tc_scale(h1p, cnt2)

    agg1 = _sc_agg(h1sp.reshape(n, h), eidx)
    h2sp = _tc_layer(agg1.reshape(NC, np2, 128), h1sp, dinvp, b1p, w2bd,
                     relu=True)

    agg2 = _sc_agg(h2sp.reshape(n, h), eidx)
    a_p, b_p = _tc_decode_tables(agg2.reshape(NC, np2, 128), h2sp, dinvp,
                                 b2p, wabd, wbbd, l1bp)

    wvec = lin2_W.reshape(1, h)
    biasvec = jnp.full((LANES,), lin2_b[0] / LANES, dtype=jnp.float32)
    out = _sc_decode(a_p.reshape(n, h), b_p.reshape(n, h), wvec, biasvec,
                     eidx)
    return out.reshape(e, 1)


# speedup vs baseline: 1.3713x; 1.0189x over previous
import functools

import jax
import jax.numpy as jnp
from jax import lax
from jax.experimental import pallas as pl
from jax.experimental.pallas import tpu as pltpu
from jax.experimental.pallas import tpu_sc as plsc

NC = 2
NS = 16
LANES = 16
NW = NC * NS


def _vector_mesh():
    return plsc.VectorSubcoreMesh(core_axis_name="c", subcore_axis_name="s")


_SC_PARAMS = pltpu.CompilerParams(use_tc_tiling_on_sc=False)
_SC_PARAMS_NOLAYOUT = pltpu.CompilerParams(use_tc_tiling_on_sc=False,
                                           needs_layout_passes=False)


def _padded_rows(n):
    return ((n + NS * 8 - 1) // (NS * 8)) * (NS * 8)


def _sc_counts(eidx, n, h):
    _, _, C, K = eidx.shape
    np_ = _padded_rows(n)
    rpt = np_ // NS

    @functools.partial(
        pl.kernel,
        out_type=jax.ShapeDtypeStruct((NC, np_, h), jnp.float32),
        mesh=_vector_mesh(),
        compiler_params=_SC_PARAMS,
        scratch_types=[
            pltpu.VMEM((C, K), jnp.int32),
            pltpu.VMEM((K, h), jnp.float32),
            pltpu.VMEM((rpt, h), jnp.float32),
            pltpu.VMEM_SHARED((np_, h), jnp.float32),
            pltpu.SemaphoreType.DMA,
            pltpu.SemaphoreType.DMA,
        ],
    )
    def k(eidx_hbm, out_hbm, idx_v, ones_v, zero_v, acc_s, sem0, sem1):
        cid = lax.axis_index("c")
        sid = lax.axis_index("s")
        wid = sid * NC + cid

        pltpu.sync_copy(eidx_hbm.at[1, wid], idx_v)

        @pl.loop(0, K)
        def _(i):
            @pl.loop(0, h, step=LANES)
            def _(c):
                ones_v[i, pl.ds(c, LANES)] = jnp.ones((LANES,), jnp.float32)

        @pl.loop(0, rpt)
        def _(i):
            @pl.loop(0, h, step=LANES)
            def _(c):
                zero_v[i, pl.ds(c, LANES)] = jnp.zeros((LANES,), jnp.float32)

        base = sid * rpt
        pltpu.sync_copy(zero_v, acc_s.at[pl.ds(base, rpt)])
        plsc.subcore_barrier()

        sems = [sem0, sem1]

        def scatter(j, b):
            pltpu.async_copy(ones_v, acc_s.at[idx_v.at[j]], sems[b],
                             add=True)

        def wait_scatter(b):
            pltpu.make_async_copy(ones_v, acc_s.at[idx_v.at[0]],
                                  sems[b]).wait()

        scatter(0, 0)
        scatter(1, 1)

        @pl.loop(0, (C - 1) // 2)
        def _(m):
            j = 2 * m
            for b in range(2):
                wait_scatter(b)

                @pl.when(j + b + 2 < C)
                def _():
                    scatter(j + b + 2, b)

        wait_scatter(0)

        plsc.subcore_barrier()
        pltpu.sync_copy(acc_s.at[pl.ds(base, rpt)],
                        out_hbm.at[cid, pl.ds(base, rpt)])

    return k(eidx)


def _sc_agg(table, eidx):
    n, h = table.shape
    _, _, C, K = eidx.shape
    np_ = _padded_rows(n)
    rpt = np_ // NS

    nbuf = 8

    @functools.partial(
        pl.kernel,
        out_type=jax.ShapeDtypeStruct((NC, np_, h), jnp.float32),
        mesh=_vector_mesh(),
        compiler_params=_SC_PARAMS,
        scratch_types=[
            pltpu.VMEM((2, C, K), jnp.int32),
            pltpu.VMEM((nbuf, K, h), jnp.float32),
            pltpu.VMEM((rpt // 4, h), jnp.float32),
            pltpu.VMEM_SHARED((np_, h), jnp.float32),
            [pltpu.SemaphoreType.DMA] * nbuf,
            [pltpu.SemaphoreType.DMA] * nbuf,
            pltpu.SemaphoreType.DMA,
        ],
    )
    def k(table_hbm, eidx_hbm, out_hbm, idx_v, bufs, zero_v,
          acc_s, gsems, ssems, isem):
        cid = lax.axis_index("c")
        sid = lax.axis_index("s")
        wid = sid * NC + cid

        cp_s = pltpu.async_copy(eidx_hbm.at[0, wid], idx_v.at[0], isem)
        cp_d = pltpu.async_copy(eidx_hbm.at[1, wid], idx_v.at[1], isem)

        rq = rpt // 4

        @pl.loop(0, rq)
        def _(i):
            @pl.loop(0, h, step=LANES)
            def _(c):
                zero_v[i, pl.ds(c, LANES)] = jnp.zeros((LANES,), jnp.float32)

        base = sid * rpt
        cp_s.wait()
        cp_d.wait()
        for t in range(4):
            pltpu.sync_copy(zero_v, acc_s.at[pl.ds(base + t * rq, rq)])
        plsc.subcore_barrier()

        def gather(j, b):
            pltpu.async_copy(table_hbm.at[idx_v.at[0, j]], bufs.at[b],
                             gsems[b])

        def wait_gather(b):
            pltpu.make_async_copy(table_hbm.at[idx_v.at[0, 0]], bufs.at[b],
                                  gsems[b]).wait()

        def scatter(j, b):
            pltpu.async_copy(bufs.at[b], acc_s.at[idx_v.at[1, j]], ssems[b],
                             add=True)

        def wait_scatter(b):
            pltpu.make_async_copy(bufs.at[b], acc_s.at[idx_v.at[1, 0]],
                                  ssems[b]).wait()

        for b in range(nbuf):
            gather(b, b)

        @pl.loop(0, (C + nbuf - 1) // nbuf)
        def _(m):
            j0 = nbuf * m
            for b in range(nbuf):
                j = j0 + b

                @pl.when(j < C)
                def _():
                    wait_gather(b)
                    scatter(j, b)

                    @pl.when(j + nbuf < C)
                    def _():
                        wait_scatter(b)
                        gather(j + nbuf, b)

        for b in range(nbuf):
            wait_scatter(b)

        plsc.subcore_barrier()
        pltpu.sync_copy(acc_s.at[pl.ds(base, rpt)],
                        out_hbm.at[cid, pl.ds(base, rpt)])

    return k(table, eidx)


def _sc_decode(a_tab, b_tab, wvec, biasvec, eidx):
    n, h = a_tab.shape
    _, _, C, K = eidx.shape
    ew = C * K
    e = NW * ew
    nslice = h // LANES
    ngrp = K // LANES
    nbuf = 2
    assert (C - 1) % nbuf == 0

    @functools.partial(
        pl.kernel,
        out_type=jax.ShapeDtypeStruct((e,), jnp.float32),
        mesh=_vector_mesh(),
        compiler_params=_SC_PARAMS_NOLAYOUT,
        scratch_types=[
            pltpu.VMEM((2, C, K), jnp.int32),
            pltpu.VMEM((nbuf, K, h), jnp.float32),
            pltpu.VMEM((nbuf, K, h), jnp.float32),
            pltpu.VMEM((nbuf, K), jnp.float32),
            pltpu.VMEM((h,), jnp.float32),
            pltpu.VMEM((LANES,), jnp.float32),
            [pltpu.SemaphoreType.DMA] * (2 * nbuf),
        ],
    )
    def k(a_hbm, b_hbm, w_hbm, bias_hbm, eidx_hbm, out_hbm, idx_v,
          bufs_a, bufs_b, out_v, w_v, bias_v, gsems):
        cid = lax.axis_index("c")
        sid = lax.axis_index("s")
        wid = sid * NC + cid

        pltpu.sync_copy(eidx_hbm.at[0, wid], idx_v.at[0])
        pltpu.sync_copy(eidx_hbm.at[1, wid], idx_v.at[1])
        pltpu.sync_copy(w_hbm.at[0], w_v)
        pltpu.sync_copy(bias_hbm, bias_v)

        ws = [w_v[pl.ds(c * LANES, LANES)] for c in range(nslice)]
        bias = bias_v[...]
        lane = lax.iota(jnp.int32, LANES)
        masks = [lane == li for li in range(LANES)]

        def gather(j, slot):
            pltpu.async_copy(a_hbm.at[idx_v.at[0, j]], bufs_a.at[slot],
                             gsems[slot])
            pltpu.async_copy(b_hbm.at[idx_v.at[1, j]], bufs_b.at[slot],
                             gsems[nbuf + slot])

        def wait_gather(slot):
            pltpu.make_async_copy(a_hbm.at[idx_v.at[0, 0]], bufs_a.at[slot],
                                  gsems[slot]).wait()
            pltpu.make_async_copy(b_hbm.at[idx_v.at[1, 0]], bufs_b.at[slot],
                                  gsems[nbuf + slot]).wait()

        def compute_store(j, slot):
            a_v = bufs_a.at[slot]
            b_v = bufs_b.at[slot]
            o_v = out_v.at[slot]

            @pl.loop(0, ngrp)
            def _(g):
                logits = bias
                for li in range(LANES):
                    ei = g * LANES + li
                    acc = bias
                    for c in range(nslice):
                        s = (a_v[ei, pl.ds(c * LANES, LANES)]
                             + b_v[ei, pl.ds(c * LANES, LANES)])
                        acc = acc + ws[c] * jnp.maximum(s, 0.0)
                    tot = jnp.broadcast_to(jnp.sum(acc), (LANES,))
                    logits = jnp.where(masks[li], tot, logits)
                o_v[pl.ds(g * LANES, LANES)] = 1.0 / (1.0 + jnp.exp(-logits))

            pltpu.sync_copy(o_v, out_hbm.at[pl.ds(wid * ew + j * K, K)])

        for b in range(nbuf):
            gather(b, b)

        @pl.loop(0, (C - 1) // nbuf)
        def _(m):
            j0 = nbuf * m
            for b in range(nbuf):
                j = j0 + b
                wait_gather(b)
                compute_store(j, b)

                @pl.when(j + nbuf < C)
                def _():
                    gather(j + nbuf, b)

        wait_gather(0)
        compute_store(C - 1, 0)

    return k(a_tab, b_tab, wvec, biasvec, eidx)


def _tc_mm(xx, ww, bn=1000):
    n, d = xx.shape
    h = ww.shape[1]

    def body(x_ref, w_ref, o_ref):
        o_ref[...] = jnp.dot(x_ref[...], w_ref[...],
                             preferred_element_type=jnp.float32)

    return pl.pallas_call(
        body,
        grid=(n // bn,),
        in_specs=[pl.BlockSpec((bn, d), lambda i: (i, 0)),
                  pl.BlockSpec((d, h), lambda i: (0, 0))],
        out_specs=pl.BlockSpec((bn, h), lambda i: (i, 0)),
        out_shape=jax.ShapeDtypeStruct((n, h), jnp.float32),
    )(xx, ww)


def _tc_scale(h1p, cnt2, bn=1000):
    n2, w = h1p.shape

    def body(h_ref, c_ref, o_ref, dinv_ref):
        dinv = lax.rsqrt(1.0 + c_ref[0] + c_ref[1])
        o_ref[...] = h_ref[...] * dinv
        dinv_ref[...] = dinv

    return pl.pallas_call(
        body,
        grid=(n2 // bn,),
        in_specs=[pl.BlockSpec((bn, w), lambda i: (i, 0)),
                  pl.BlockSpec((NC, bn, w), lambda i: (0, i, 0))],
        out_specs=[pl.BlockSpec((bn, w), lambda i: (i, 0)),
                   pl.BlockSpec((bn, w), lambda i: (i, 0))],
        out_shape=[jax.ShapeDtypeStruct((n2, w), jnp.float32),
                   jax.ShapeDtypeStruct((n2, w), jnp.float32)],
    )(h1p, cnt2)


def _tc_layer(aggp, hsp, dinvp, b128, wbd, relu, bn=1000):
    n2, w = hsp.shape

    def body(a_ref, hs_ref, dinv_ref, b_ref, w_ref, o_ref):
        dinv = dinv_ref[...]
        z = dinv * (a_ref[0] + a_ref[1] + hs_ref[...]) + b_ref[...]
        if relu:
            z = jnp.maximum(z, 0.0)
        o_ref[...] = jnp.dot(z, w_ref[...],
                             preferred_element_type=jnp.float32) * dinv

    return pl.pallas_call(
        body,
        grid=(n2 // bn,),
        in_specs=[pl.BlockSpec((NC, bn, w), lambda i: (0, i, 0)),
                  pl.BlockSpec((bn, w), lambda i: (i, 0)),
                  pl.BlockSpec((bn, w), lambda i: (i, 0)),
                  pl.BlockSpec((1, w), lambda i: (0, 0)),
                  pl.BlockSpec((w, w), lambda i: (0, 0))],
        out_specs=pl.BlockSpec((bn, w), lambda i: (i, 0)),
        out_shape=jax.ShapeDtypeStruct((n2, w), jnp.float32),
    )(aggp, hsp, dinvp, b128, wbd)


def _tc_decode_tables(aggp, hsp, dinvp, b128, wabd, wbbd, l1b128, bn=1000):
    n2, w = hsp.shape

    def body(a_ref, hs_ref, dinv_ref, b_ref, wa_ref, wb_ref, l1b_ref,
             oa_ref, ob_ref):
        z = dinv_ref[...] * (a_ref[0] + a_ref[1] + hs_ref[...]) + b_ref[...]
        oa_ref[...] = jnp.dot(z, wa_ref[...],
                              preferred_element_type=jnp.float32) + l1b_ref[...]
        ob_ref[...] = jnp.dot(z, wb_ref[...],
                              preferred_element_type=jnp.float32)

    return pl.pallas_call(
        body,
        grid=(n2 // bn,),
        in_specs=[pl.BlockSpec((NC, bn, w), lambda i: (0, i, 0)),
                  pl.BlockSpec((bn, w), lambda i: (i, 0)),
                  pl.BlockSpec((bn, w), lambda i: (i, 0)),
                  pl.BlockSpec((1, w), lambda i: (0, 0)),
                  pl.BlockSpec((w, w), lambda i: (0, 0)),
                  pl.BlockSpec((w, w), lambda i: (0, 0)),
                  pl.BlockSpec((1, w), lambda i: (0, 0))],
        out_specs=[pl.BlockSpec((bn, w), lambda i: (i, 0)),
                   pl.BlockSpec((bn, w), lambda i: (i, 0))],
        out_shape=[jax.ShapeDtypeStruct((n2, w), jnp.float32),
                   jax.ShapeDtypeStruct((n2, w), jnp.float32)],
    )(aggp, hsp, dinvp, b128, wabd, wbbd, l1b128)


def kernel(x, edge_index, W1, b1, W2, b2, lin1_W, lin1_b, lin2_W, lin2_b):
    n, d = x.shape
    h = W1.shape[1]
    e = edge_index.shape[1]

    ew = e // NW
    K = 80
    C = ew // K
    assert ew == C * K and C % 2 == 1

    n2 = n // 2
    np2 = _padded_rows(n) * h // 128

    eidx = edge_index.reshape(2, NW, C, K)

    zdh = jnp.zeros((d, h), jnp.float32)
    zhh = jnp.zeros((h, h), jnp.float32)
    w1a, w1b = lin1_W[:h], lin1_W[h:]
    w1bd = jnp.block([[W1, zdh], [zdh, W1]])
    w2bd = jnp.block([[W2, zhh], [zhh, W2]])
    wabd = jnp.block([[w1a, zhh], [zhh, w1a]])
    wbbd = jnp.block([[w1b, zhh], [zhh, w1b]])
    b1p = jnp.concatenate([b1, b1]).reshape(1, 2 * h)
    b2p = jnp.concatenate([b2, b2]).reshape(1, 2 * h)
    l1bp = jnp.concatenate([lin1_b, lin1_b]).reshape(1, 2 * h)

    cnt2 = _sc_counts(eidx, n, h).reshape(NC, np2, 128)
    h1p = _tc_mm(x.reshape(n2, 2 * d), w1bd, bn=1000)
    h1sp, dinvp = _tc_scale(h1p, cnt2)

    agg1 = _sc_agg(h1sp.reshape(n, h), eidx)
    h2sp = _tc_layer(agg1.reshape(NC, np2, 128), h1sp, dinvp, b1p, w2bd,
                     relu=True)

    agg2 = _sc_agg(h2sp.reshape(n, h), eidx)
    a_p, b_p = _tc_decode_tables(agg2.reshape(NC, np2, 128), h2sp, dinvp,
                                 b2p, wabd, wbbd, l1bp)

    wvec = lin2_W.reshape(1, h)
    biasvec = jnp.full((LANES,), lin2_b[0] / LANES, dtype=jnp.float32)
    out = _sc_decode(a_p.reshape(n, h), b_p.reshape(n, h), wvec, biasvec,
                     eidx)
    return out.reshape(e, 1)


# speedup vs baseline: 1.4976x; 1.0921x over previous
import functools

import jax
import jax.numpy as jnp
from jax import lax
from jax.experimental import pallas as pl
from jax.experimental.pallas import tpu as pltpu
from jax.experimental.pallas import tpu_sc as plsc

NC = 2
NS = 16
LANES = 16
NW = NC * NS


def _vector_mesh():
    return plsc.VectorSubcoreMesh(core_axis_name="c", subcore_axis_name="s")


_SC_PARAMS = pltpu.CompilerParams(use_tc_tiling_on_sc=False)
_SC_PARAMS_NOLAYOUT = pltpu.CompilerParams(use_tc_tiling_on_sc=False,
                                           needs_layout_passes=False)


def _padded_rows(n):
    return ((n + NS * 8 - 1) // (NS * 8)) * (NS * 8)


def _sc_counts(eidx, n, h):
    _, _, C, K = eidx.shape
    np_ = _padded_rows(n)
    rpt = np_ // NS

    @functools.partial(
        pl.kernel,
        out_type=jax.ShapeDtypeStruct((NC, np_, h), jnp.float32),
        mesh=_vector_mesh(),
        compiler_params=_SC_PARAMS,
        scratch_types=[
            pltpu.VMEM((C, K), jnp.int32),
            pltpu.VMEM((K, h), jnp.float32),
            pltpu.VMEM((rpt, h), jnp.float32),
            pltpu.VMEM_SHARED((np_, h), jnp.float32),
            pltpu.SemaphoreType.DMA,
            pltpu.SemaphoreType.DMA,
        ],
    )
    def k(eidx_hbm, out_hbm, idx_v, ones_v, zero_v, acc_s, sem0, sem1):
        cid = lax.axis_index("c")
        sid = lax.axis_index("s")
        wid = sid * NC + cid

        pltpu.sync_copy(eidx_hbm.at[1, wid], idx_v)

        @pl.loop(0, K)
        def _(i):
            @pl.loop(0, h, step=LANES)
            def _(c):
                ones_v[i, pl.ds(c, LANES)] = jnp.ones((LANES,), jnp.float32)

        @pl.loop(0, rpt)
        def _(i):
            @pl.loop(0, h, step=LANES)
            def _(c):
                zero_v[i, pl.ds(c, LANES)] = jnp.zeros((LANES,), jnp.float32)

        base = sid * rpt
        pltpu.sync_copy(zero_v, acc_s.at[pl.ds(base, rpt)])
        plsc.subcore_barrier()

        sems = [sem0, sem1]

        def scatter(j, b):
            pltpu.async_copy(ones_v, acc_s.at[idx_v.at[j]], sems[b],
                             add=True)

        def wait_scatter(b):
            pltpu.make_async_copy(ones_v, acc_s.at[idx_v.at[0]],
                                  sems[b]).wait()

        scatter(0, 0)
        scatter(1, 1)

        @pl.loop(0, (C - 1) // 2)
        def _(m):
            j = 2 * m
            for b in range(2):
                wait_scatter(b)

                @pl.when(j + b + 2 < C)
                def _():
                    scatter(j + b + 2, b)

        wait_scatter(0)

        plsc.subcore_barrier()
        pltpu.sync_copy(acc_s.at[pl.ds(base, rpt)],
                        out_hbm.at[cid, pl.ds(base, rpt)])

    return k(eidx)


def _sc_agg(table, eidx):
    n, h = table.shape
    _, _, C, K = eidx.shape
    np_ = _padded_rows(n)
    rpt = np_ // NS

    nbuf = 8

    @functools.partial(
        pl.kernel,
        out_type=jax.ShapeDtypeStruct((NC, np_, h), jnp.float32),
        mesh=_vector_mesh(),
        compiler_params=_SC_PARAMS,
        scratch_types=[
            pltpu.VMEM((2, C, K), jnp.int32),
            pltpu.VMEM((nbuf, K, h), jnp.float32),
            pltpu.VMEM((rpt // 4, h), jnp.float32),
            pltpu.VMEM_SHARED((np_, h), jnp.float32),
            [pltpu.SemaphoreType.DMA] * nbuf,
            [pltpu.SemaphoreType.DMA] * nbuf,
            pltpu.SemaphoreType.DMA,
        ],
    )
    def k(table_hbm, eidx_hbm, out_hbm, idx_v, bufs, zero_v,
          acc_s, gsems, ssems, isem):
        cid = lax.axis_index("c")
        sid = lax.axis_index("s")
        wid = sid * NC + cid

        cp_s = pltpu.async_copy(eidx_hbm.at[0, wid], idx_v.at[0], isem)
        cp_d = pltpu.async_copy(eidx_hbm.at[1, wid], idx_v.at[1], isem)

        rq = rpt // 4

        @pl.loop(0, rq)
        def _(i):
            @pl.loop(0, h, step=LANES)
            def _(c):
                zero_v[i, pl.ds(c, LANES)] = jnp.zeros((LANES,), jnp.float32)

        base = sid * rpt
        cp_s.wait()
        cp_d.wait()
        for t in range(4):
            pltpu.sync_copy(zero_v, acc_s.at[pl.ds(base + t * rq, rq)])
        plsc.subcore_barrier()

        def gather(j, b):
            pltpu.async_copy(table_hbm.at[idx_v.at[0, j]], bufs.at[b],
                             gsems[b])

        def wait_gather(b):
            pltpu.make_async_copy(table_hbm.at[idx_v.at[0, 0]], bufs.at[b],
                                  gsems[b]).wait()

        def scatter(j, b):
            pltpu.async_copy(bufs.at[b], acc_s.at[idx_v.at[1, j]], ssems[b],
                             add=True)

        def wait_scatter(b):
            pltpu.make_async_copy(bufs.at[b], acc_s.at[idx_v.at[1, 0]],
                                  ssems[b]).wait()

        for b in range(nbuf):
            gather(b, b)

        @pl.loop(0, (C + nbuf - 1) // nbuf)
        def _(m):
            j0 = nbuf * m
            for b in range(nbuf):
                j = j0 + b

                @pl.when(j < C)
                def _():
                    wait_gather(b)
                    scatter(j, b)

                    @pl.when(j + nbuf < C)
                    def _():
                        wait_scatter(b)
                        gather(j + nbuf, b)

        for b in range(nbuf):
            wait_scatter(b)

        plsc.subcore_barrier()
        pltpu.sync_copy(acc_s.at[pl.ds(base, rpt)],
                        out_hbm.at[cid, pl.ds(base, rpt)])

    return k(table, eidx)


def _sc_decode(a_tab, b_tab, wvec, biasvec, eidx):
    n, h = a_tab.shape
    _, _, C, K = eidx.shape
    ew = C * K
    e = NW * ew
    bl = 2 * LANES
    nslice = h // bl
    ngrp = K // LANES
    nbuf = 2
    assert (C - 1) % nbuf == 0

    @functools.partial(
        pl.kernel,
        out_type=jax.ShapeDtypeStruct((e,), jnp.float32),
        mesh=_vector_mesh(),
        compiler_params=_SC_PARAMS_NOLAYOUT,
        scratch_types=[
            pltpu.VMEM((2, C, K), jnp.int32),
            pltpu.VMEM((nbuf, K, h), jnp.bfloat16),
            pltpu.VMEM((nbuf, K, h), jnp.bfloat16),
            pltpu.VMEM((nbuf, K), jnp.float32),
            pltpu.VMEM((h,), jnp.bfloat16),
            pltpu.VMEM((LANES,), jnp.float32),
            [pltpu.SemaphoreType.DMA] * (2 * nbuf),
        ],
    )
    def k(a_hbm, b_hbm, w_hbm, bias_hbm, eidx_hbm, out_hbm, idx_v,
          bufs_a, bufs_b, out_v, w_v, bias_v, gsems):
        cid = lax.axis_index("c")
        sid = lax.axis_index("s")
        wid = sid * NC + cid

        pltpu.sync_copy(eidx_hbm.at[0, wid], idx_v.at[0])
        pltpu.sync_copy(eidx_hbm.at[1, wid], idx_v.at[1])
        pltpu.sync_copy(w_hbm.at[0], w_v)
        pltpu.sync_copy(bias_hbm, bias_v)

        ws = [w_v[pl.ds(c * bl, bl)] for c in range(nslice)]
        bias = bias_v[...]
        lane = lax.iota(jnp.int32, LANES)
        masks = [lane == li for li in range(LANES)]

        def gather(j, slot):
            pltpu.async_copy(a_hbm.at[idx_v.at[0, j]], bufs_a.at[slot],
                             gsems[slot])
            pltpu.async_copy(b_hbm.at[idx_v.at[1, j]], bufs_b.at[slot],
                             gsems[nbuf + slot])

        def wait_gather(slot):
            pltpu.make_async_copy(a_hbm.at[idx_v.at[0, 0]], bufs_a.at[slot],
                                  gsems[slot]).wait()
            pltpu.make_async_copy(b_hbm.at[idx_v.at[1, 0]], bufs_b.at[slot],
                                  gsems[nbuf + slot]).wait()

        def compute_store(j, slot):
            a_v = bufs_a.at[slot]
            b_v = bufs_b.at[slot]
            o_v = out_v.at[slot]

            @pl.loop(0, ngrp)
            def _(g):
                logits = bias
                for li in range(LANES):
                    ei = g * LANES + li
                    acc = None
                    for c in range(nslice):
                        s = (a_v[ei, pl.ds(c * bl, bl)]
                             + b_v[ei, pl.ds(c * bl, bl)])
                        t = ws[c] * jnp.maximum(s, jnp.bfloat16(0.0))
                        acc = t if acc is None else acc + t
                    u0, u1 = plsc.unpack(acc, format=plsc.PackFormat.INTERLEAVED)
                    tot = jnp.broadcast_to(jnp.sum(u0 + u1 + bias), (LANES,))
                    logits = jnp.where(masks[li], tot, logits)
                o_v[pl.ds(g * LANES, LANES)] = 1.0 / (1.0 + jnp.exp(-logits))

            pltpu.sync_copy(o_v, out_hbm.at[pl.ds(wid * ew + j * K, K)])

        for b in range(nbuf):
            gather(b, b)

        @pl.loop(0, (C - 1) // nbuf)
        def _(m):
            j0 = nbuf * m
            for b in range(nbuf):
                j = j0 + b
                wait_gather(b)
                compute_store(j, b)

                @pl.when(j + nbuf < C)
                def _():
                    gather(j + nbuf, b)

        wait_gather(0)
        compute_store(C - 1, 0)

    return k(a_tab, b_tab, wvec, biasvec, eidx)


def _tc_mm(xx, ww, bn=1000):
    n, d = xx.shape
    h = ww.shape[1]

    def body(x_ref, w_ref, o_ref):
        o_ref[...] = jnp.dot(x_ref[...], w_ref[...],
                             preferred_element_type=jnp.float32)

    return pl.pallas_call(
        body,
        grid=(n // bn,),
        in_specs=[pl.BlockSpec((bn, d), lambda i: (i, 0)),
                  pl.BlockSpec((d, h), lambda i: (0, 0))],
        out_specs=pl.BlockSpec((bn, h), lambda i: (i, 0)),
        out_shape=jax.ShapeDtypeStruct((n, h), jnp.float32),
    )(xx, ww)


def _tc_scale(h1p, cnt2, bn=1000):
    n2, w = h1p.shape

    def body(h_ref, c_ref, o_ref, dinv_ref):
        dinv = lax.rsqrt(1.0 + c_ref[0] + c_ref[1])
        o_ref[...] = h_ref[...] * dinv
        dinv_ref[...] = dinv

    return pl.pallas_call(
        body,
        grid=(n2 // bn,),
        in_specs=[pl.BlockSpec((bn, w), lambda i: (i, 0)),
                  pl.BlockSpec((NC, bn, w), lambda i: (0, i, 0))],
        out_specs=[pl.BlockSpec((bn, w), lambda i: (i, 0)),
                   pl.BlockSpec((bn, w), lambda i: (i, 0))],
        out_shape=[jax.ShapeDtypeStruct((n2, w), jnp.float32),
                   jax.ShapeDtypeStruct((n2, w), jnp.float32)],
    )(h1p, cnt2)


def _tc_layer(aggp, hsp, dinvp, b128, wbd, relu, bn=1000):
    n2, w = hsp.shape

    def body(a_ref, hs_ref, dinv_ref, b_ref, w_ref, o_ref):
        dinv = dinv_ref[...]
        z = dinv * (a_ref[0] + a_ref[1] + hs_ref[...]) + b_ref[...]
        if relu:
            z = jnp.maximum(z, 0.0)
        o_ref[...] = jnp.dot(z, w_ref[...],
                             preferred_element_type=jnp.float32) * dinv

    return pl.pallas_call(
        body,
        grid=(n2 // bn,),
        in_specs=[pl.BlockSpec((NC, bn, w), lambda i: (0, i, 0)),
                  pl.BlockSpec((bn, w), lambda i: (i, 0)),
                  pl.BlockSpec((bn, w), lambda i: (i, 0)),
                  pl.BlockSpec((1, w), lambda i: (0, 0)),
                  pl.BlockSpec((w, w), lambda i: (0, 0))],
        out_specs=pl.BlockSpec((bn, w), lambda i: (i, 0)),
        out_shape=jax.ShapeDtypeStruct((n2, w), jnp.float32),
    )(aggp, hsp, dinvp, b128, wbd)


def _tc_decode_tables(aggp, hsp, dinvp, b128, wabd, wbbd, l1b128, bn=1000):
    n2, w = hsp.shape

    def body(a_ref, hs_ref, dinv_ref, b_ref, wa_ref, wb_ref, l1b_ref,
             oa_ref, ob_ref):
        z = dinv_ref[...] * (a_ref[0] + a_ref[1] + hs_ref[...]) + b_ref[...]
        oa_ref[...] = (jnp.dot(z, wa_ref[...], preferred_element_type=jnp.float32)
                       + l1b_ref[...]).astype(jnp.bfloat16)
        ob_ref[...] = jnp.dot(z, wb_ref[...],
                              preferred_element_type=jnp.float32
                              ).astype(jnp.bfloat16)

    return pl.pallas_call(
        body,
        grid=(n2 // bn,),
        in_specs=[pl.BlockSpec((NC, bn, w), lambda i: (0, i, 0)),
                  pl.BlockSpec((bn, w), lambda i: (i, 0)),
                  pl.BlockSpec((bn, w), lambda i: (i, 0)),
                  pl.BlockSpec((1, w), lambda i: (0, 0)),
                  pl.BlockSpec((w, w), lambda i: (0, 0)),
                  pl.BlockSpec((w, w), lambda i: (0, 0)),
                  pl.BlockSpec((1, w), lambda i: (0, 0))],
        out_specs=[pl.BlockSpec((bn, w), lambda i: (i, 0)),
                   pl.BlockSpec((bn, w), lambda i: (i, 0))],
        out_shape=[jax.ShapeDtypeStruct((n2, w), jnp.bfloat16),
                   jax.ShapeDtypeStruct((n2, w), jnp.bfloat16)],
    )(aggp, hsp, dinvp, b128, wabd, wbbd, l1b128)


def kernel(x, edge_index, W1, b1, W2, b2, lin1_W, lin1_b, lin2_W, lin2_b):
    n, d = x.shape
    h = W1.shape[1]
    e = edge_index.shape[1]

    ew = e // NW
    K = 80
    C = ew // K
    assert ew == C * K and C % 2 == 1

    n2 = n // 2
    np2 = _padded_rows(n) * h // 128

    eidx = edge_index.reshape(2, NW, C, K)

    zdh = jnp.zeros((d, h), jnp.float32)
    zhh = jnp.zeros((h, h), jnp.float32)
    w1a, w1b = lin1_W[:h], lin1_W[h:]
    w1bd = jnp.block([[W1, zdh], [zdh, W1]])
    w2bd = jnp.block([[W2, zhh], [zhh, W2]])
    wabd = jnp.block([[w1a, zhh], [zhh, w1a]])
    wbbd = jnp.block([[w1b, zhh], [zhh, w1b]])
    b1p = jnp.concatenate([b1, b1]).reshape(1, 2 * h)
    b2p = jnp.concatenate([b2, b2]).reshape(1, 2 * h)
    l1bp = jnp.concatenate([lin1_b, lin1_b]).reshape(1, 2 * h)

    cnt2 = _sc_counts(eidx, n, h).reshape(NC, np2, 128)
    h1p = _tc_mm(x.reshape(n2, 2 * d), w1bd, bn=1000)
    h1sp, dinvp = _tc_scale(h1p, cnt2)

    agg1 = _sc_agg(h1sp.reshape(n, h), eidx)
    h2sp = _tc_layer(agg1.reshape(NC, np2, 128), h1sp, dinvp, b1p, w2bd,
                     relu=True)

    agg2 = _sc_agg(h2sp.reshape(n, h), eidx)
    a_p, b_p = _tc_decode_tables(agg2.reshape(NC, np2, 128), h2sp, dinvp,
                                 b2p, wabd, wbbd, l1bp)

    wvec = lin2_W.astype(jnp.bfloat16).reshape(1, h)
    biasvec = jnp.full((LANES,), lin2_b[0] / LANES, dtype=jnp.float32)
    out = _sc_decode(a_p.reshape(n, h), b_p.reshape(n, h), wvec, biasvec,
                     eidx)
    return out.reshape(e, 1)
